# baseline (speedup 1.0000x reference)
import functools

import jax
import jax.numpy as jnp
from jax import lax
from jax.experimental import pallas as pl
from jax.experimental.pallas import tpu as pltpu
from jax.experimental.pallas import tpu_sc as plsc

D = 64
SCALE = 8.0
NC, NS, L = 2, 16, 16
NW = NC * NS
TB = 128


def _make_kernel(T, S):
    n_tb = T // (NW * TB)
    n_units = S * n_tb
    mesh = plsc.VectorSubcoreMesh(core_axis_name="c", subcore_axis_name="s")

    @functools.partial(
        pl.kernel,
        mesh=mesh,
        compiler_params=pltpu.CompilerParams(needs_layout_passes=False),
        out_type=jax.ShapeDtypeStruct((S, D, T), jnp.float32),
        scratch_types=[
            pltpu.VMEM((TB,), jnp.int32),
            pltpu.VMEM((TB,), jnp.int32),
            pltpu.VMEM((TB, 2 * D), jnp.float32),
            pltpu.VMEM((D, TB), jnp.float32),
            pltpu.SemaphoreType.DMA,
        ],
    )
    def k(tok_t, tt, out, tok_v, idx_v, rows_v, obuf, gsem):
        wid = lax.axis_index("s") * NC + lax.axis_index("c")
        t_base = wid * (n_tb * TB)

        def unit(u, carry):
            s = u // n_tb
            t0 = t_base + (u % n_tb) * TB
            pltpu.sync_copy(tok_t.at[s, pl.ds(t0, TB)], tok_v)
            for i in range(TB // L):
                sl = pl.ds(i * L, L)
                idx_v[sl] = lax.shift_right_logical(tok_v[sl], 1)
            pltpu.async_copy(tt.at[idx_v], rows_v, gsem).wait()

            def dcol(j, carry2):
                d = j // (TB // L)
                lb = j % (TB // L)
                tok16 = tok_v[pl.ds(lb * L, L)]
                col = lax.shift_left(lax.bitwise_and(tok16, 1), 6) + d
                row = lax.iota(jnp.int32, L) + lb * L
                vals = plsc.load_gather(rows_v, [row, col])
                obuf[d, pl.ds(lb * L, L)] = vals * SCALE
                return carry2

            lax.fori_loop(0, D * (TB // L), dcol, 0, unroll=8)
            pltpu.sync_copy(obuf, out.at[s, :, pl.ds(t0, TB)])
            return carry

        lax.fori_loop(0, n_units, unit, 0)

    return k


def kernel(tokens, table):
    T, S = tokens.shape
    V = table.shape[0]
    tt = table.reshape(V // 2, 2 * D)
    out_t = _make_kernel(T, S)(tokens.T, tt)
    return out_t.transpose(2, 0, 1)


# speedup vs baseline: 1.3157x; 1.3157x over previous
import functools

import jax
import jax.numpy as jnp
from jax import lax
from jax.experimental import pallas as pl
from jax.experimental.pallas import tpu as pltpu
from jax.experimental.pallas import tpu_sc as plsc

D = 64
SCALE = 8.0
NC, NS, L = 2, 16, 16
NW = NC * NS
TB = 128


def _make_kernel(T, S):
    n_tb = T // (NW * TB)
    n_units = S * n_tb
    mesh = plsc.VectorSubcoreMesh(core_axis_name="c", subcore_axis_name="s")

    @functools.partial(
        pl.kernel,
        mesh=mesh,
        compiler_params=pltpu.CompilerParams(needs_layout_passes=False),
        out_type=jax.ShapeDtypeStruct((S, D, T), jnp.float32),
        scratch_types=[
            pltpu.VMEM((TB,), jnp.int32),
            pltpu.VMEM((TB,), jnp.int32),
            pltpu.VMEM((TB, 2 * D), jnp.float32),
            pltpu.VMEM((D, TB), jnp.float32),
            pltpu.SemaphoreType.DMA,
        ],
    )
    def k(tok_t, tt, out, tok_v, idx_v, rows_v, obuf, gsem):
        wid = lax.axis_index("s") * NC + lax.axis_index("c")
        t_base = wid * (n_tb * TB)

        def unit(u, carry):
            s = u // n_tb
            t0 = t_base + (u % n_tb) * TB
            pltpu.sync_copy(tok_t.at[s, pl.ds(t0, TB)], tok_v)
            for i in range(TB // L):
                sl = pl.ds(i * L, L)
                idx_v[sl] = lax.shift_right_logical(tok_v[sl], 1)
            pltpu.async_copy(tt.at[idx_v], rows_v, gsem).wait()

            for lb in range(TB // L):
                tok16 = tok_v[pl.ds(lb * L, L)]
                colbase = lax.shift_left(lax.bitwise_and(tok16, 1), 6)
                row = lax.iota(jnp.int32, L) + lb * L

                def dcol(dd, carry2, colbase=colbase, row=row, lb=lb):
                    vals = plsc.load_gather(rows_v, [row, colbase + dd])
                    obuf[dd, pl.ds(lb * L, L)] = vals * SCALE
                    return carry2

                lax.fori_loop(0, D, dcol, 0, unroll=8)
            pltpu.sync_copy(obuf, out.at[s, :, pl.ds(t0, TB)])
            return carry

        lax.fori_loop(0, n_units, unit, 0)

    return k


def kernel(tokens, table):
    T, S = tokens.shape
    V = table.shape[0]
    tt = table.reshape(V // 2, 2 * D)
    out_t = _make_kernel(T, S)(tokens.T, tt)
    return out_t.transpose(2, 0, 1)


# speedup vs baseline: 1.5202x; 1.1555x over previous
import functools

import jax
import jax.numpy as jnp
from jax import lax
from jax.experimental import pallas as pl
from jax.experimental.pallas import tpu as pltpu
from jax.experimental.pallas import tpu_sc as plsc

D = 64
SCALE = 8.0
NC, NS, L = 2, 16, 16
NW = NC * NS
TB = 128
RP = 2 * D + 1


def _make_kernel(T, S):
    n_tb = T // (NW * TB)
    n_units = S * n_tb
    assert n_units % 2 == 0 and n_units >= 4
    mesh = plsc.VectorSubcoreMesh(core_axis_name="c", subcore_axis_name="s")

    @functools.partial(
        pl.kernel,
        mesh=mesh,
        compiler_params=pltpu.CompilerParams(needs_layout_passes=False),
        out_type=jax.ShapeDtypeStruct((S, D, T), jnp.float32),
        scratch_types=[
            pltpu.VMEM((TB,), jnp.int32),
            pltpu.VMEM((TB,), jnp.int32),
            pltpu.VMEM((TB,), jnp.int32),
            pltpu.VMEM((TB,), jnp.int32),
            pltpu.VMEM((TB,), jnp.int32),
            pltpu.VMEM((TB,), jnp.int32),
            pltpu.VMEM((TB, RP), jnp.float32),
            pltpu.VMEM((TB, RP), jnp.float32),
            pltpu.VMEM((D, TB), jnp.float32),
            pltpu.SemaphoreType.DMA,
            pltpu.SemaphoreType.DMA,
            pltpu.SemaphoreType.DMA,
            pltpu.SemaphoreType.DMA,
        ],
    )
    def k(tok_t, tt, out, tokA, tokB, idxA, idxB, cbA, cbB,
          rowsA, rowsB, obuf, tsA, tsB, gsA, gsB):
        wid = lax.axis_index("s") * NC + lax.axis_index("c")
        t_base = wid * (n_tb * TB)

        def tok_src(u):
            s = u // n_tb
            t0 = t_base + (u % n_tb) * TB
            return tok_t.at[s, pl.ds(t0, TB)]

        def start_tok(u, tok_v, sem):
            return pltpu.async_copy(tok_src(u), tok_v, sem)

        def wait_tok(u, tok_v, sem):
            pltpu.make_async_copy(tok_src(u), tok_v, sem).wait()

        def compute_idx(tok_v, idx_v, cb_v):
            for i in range(TB // L):
                sl = pl.ds(i * L, L)
                v = tok_v[sl]
                idx_v[sl] = lax.shift_right_logical(v, 1)
                cb_v[sl] = lax.shift_left(lax.bitwise_and(v, 1), 6)

        def start_gather(idx_v, rows_v, sem):
            return pltpu.async_copy(
                tt.at[idx_v], rows_v.at[:, pl.ds(0, 2 * D)], sem)

        def wait_gather(idx_v, rows_v, sem):
            pltpu.make_async_copy(
                tt.at[idx_v], rows_v.at[:, pl.ds(0, 2 * D)], sem).wait()

        def emit_unit(u, rows_v, cb_v):
            for lb in range(TB // L):
                cb16 = cb_v[pl.ds(lb * L, L)]
                row = lax.iota(jnp.int32, L) + lb * L

                def dcol(dd, carry2, cb16=cb16, row=row, lb=lb):
                    vals = plsc.load_gather(rows_v, [row, cb16 + dd])
                    obuf[dd, pl.ds(lb * L, L)] = vals * SCALE
                    return carry2

                lax.fori_loop(0, D, dcol, 0, unroll=8)
            s = u // n_tb
            t0 = t_base + (u % n_tb) * TB
            pltpu.sync_copy(obuf, out.at[s, :, pl.ds(t0, TB)])

        pltpu.sync_copy(tok_src(0), tokA)
        compute_idx(tokA, idxA, cbA)
        start_gather(idxA, rowsA, gsA)
        start_tok(1, tokB, tsB)

        def pair(kk, carry):
            u0 = 2 * kk
            wait_tok(u0 + 1, tokB, tsB)
            compute_idx(tokB, idxB, cbB)
            start_tok(u0 + 2, tokA, tsA)
            wait_gather(idxA, rowsA, gsA)
            start_gather(idxB, rowsB, gsB)
            emit_unit(u0, rowsA, cbA)
            wait_tok(u0 + 2, tokA, tsA)
            compute_idx(tokA, idxA, cbA)
            start_tok(u0 + 3, tokB, tsB)
            wait_gather(idxB, rowsB, gsB)
            start_gather(idxA, rowsA, gsA)
            emit_unit(u0 + 1, rowsB, cbB)
            return carry

        lax.fori_loop(0, n_units // 2 - 1, pair, 0)

        u0 = n_units - 2
        wait_tok(u0 + 1, tokB, tsB)
        compute_idx(tokB, idxB, cbB)
        wait_gather(idxA, rowsA, gsA)
        start_gather(idxB, rowsB, gsB)
        emit_unit(u0, rowsA, cbA)
        wait_gather(idxB, rowsB, gsB)
        emit_unit(u0 + 1, rowsB, cbB)

    return k


def kernel(tokens, table):
    T, S = tokens.shape
    V = table.shape[0]
    tt = table.reshape(V // 2, 2 * D)
    out_t = _make_kernel(T, S)(tokens.T, tt)
    return out_t.transpose(2, 0, 1)


# speedup vs baseline: 1.6921x; 1.1131x over previous
import functools

import jax
import jax.numpy as jnp
from jax import lax
from jax.experimental import pallas as pl
from jax.experimental.pallas import tpu as pltpu
from jax.experimental.pallas import tpu_sc as plsc

D = 64
SCALE = 8.0
NC, NS, L = 2, 16, 16
NW = NC * NS
TB = 128


def _make_kernel(T, S):
    n_tb = T // (NW * TB)
    n_units = S * n_tb
    assert n_units % 2 == 0 and n_units >= 4
    mesh = plsc.VectorSubcoreMesh(core_axis_name="c", subcore_axis_name="s")

    @functools.partial(
        pl.kernel,
        mesh=mesh,
        compiler_params=pltpu.CompilerParams(needs_layout_passes=False),
        out_type=jax.ShapeDtypeStruct((S, T, 2 * D), jnp.float32),
        scratch_types=[
            pltpu.VMEM((TB,), jnp.int32),
            pltpu.VMEM((TB,), jnp.int32),
            pltpu.VMEM((TB, 2 * D), jnp.float32),
            pltpu.VMEM((TB, 2 * D), jnp.float32),
            pltpu.SemaphoreType.DMA,
            pltpu.SemaphoreType.DMA,
            pltpu.SemaphoreType.DMA,
            pltpu.SemaphoreType.DMA,
        ],
    )
    def k(tok_t, tabp, mid, tokA, tokB, rowsA, rowsB, tsA, tsB, gsA, gsB):
        wid = lax.axis_index("s") * NC + lax.axis_index("c")
        t_base = wid * (n_tb * TB)

        def tok_src(u):
            s = u // n_tb
            t0 = t_base + (u % n_tb) * TB
            return tok_t.at[s, pl.ds(t0, TB)]

        def start_tok(u, tok_v, sem):
            pltpu.async_copy(tok_src(u), tok_v, sem)

        def wait_tok(u, tok_v, sem):
            pltpu.make_async_copy(tok_src(u), tok_v, sem).wait()

        def start_gather(idx_v, rows_v, sem):
            pltpu.async_copy(tabp.at[idx_v], rows_v, sem)

        def wait_gather(idx_v, rows_v, sem):
            pltpu.make_async_copy(tabp.at[idx_v], rows_v, sem).wait()

        def emit_unit(u, rows_v):
            def srow(r, carry2):
                for kk in range(D // L):
                    sl = pl.ds(kk * L, L)
                    rows_v[r, sl] = rows_v[r, sl] * SCALE
                return carry2

            lax.fori_loop(0, TB, srow, 0, unroll=8)
            s = u // n_tb
            t0 = t_base + (u % n_tb) * TB
            pltpu.sync_copy(rows_v, mid.at[s, pl.ds(t0, TB), :])

        pltpu.sync_copy(tok_src(0), tokA)
        start_gather(tokA, rowsA, gsA)
        start_tok(1, tokB, tsB)

        def pair(kk, carry):
            u0 = 2 * kk
            wait_tok(u0 + 1, tokB, tsB)
            wait_gather(tokA, rowsA, gsA)
            start_gather(tokB, rowsB, gsB)
            start_tok(u0 + 2, tokA, tsA)
            emit_unit(u0, rowsA)
            wait_tok(u0 + 2, tokA, tsA)
            wait_gather(tokB, rowsB, gsB)
            start_gather(tokA, rowsA, gsA)
            start_tok(u0 + 3, tokB, tsB)
            emit_unit(u0 + 1, rowsB)
            return carry

        lax.fori_loop(0, n_units // 2 - 1, pair, 0)

        wait_tok(n_units - 1, tokB, tsB)
        wait_gather(tokA, rowsA, gsA)
        start_gather(tokB, rowsB, gsB)
        emit_unit(n_units - 2, rowsA)
        wait_gather(tokB, rowsB, gsB)
        emit_unit(n_units - 1, rowsB)

    return k


BT = 512


def _tc_transpose(S, T):
    def body(mid_ref, out_ref):
        out_ref[0] = jnp.transpose(mid_ref[0], (1, 0))[:D, :]

    return pl.pallas_call(
        body,
        grid=(S, T // BT),
        in_specs=[pl.BlockSpec((1, BT, 128), lambda s, tb: (s, tb, 0))],
        out_specs=pl.BlockSpec((1, D, BT), lambda s, tb: (s, 0, tb)),
        out_shape=jax.ShapeDtypeStruct((S, D, T), jnp.float32),
    )


def kernel(tokens, table):
    T, S = tokens.shape
    tabp = jnp.pad(table, ((0, 0), (0, D)))
    mid = _make_kernel(T, S)(tokens.T, tabp)
    out_t = _tc_transpose(S, T)(mid)
    return out_t.transpose(2, 0, 1)
